# baseline (speedup 1.0000x reference)
import jax
import jax.numpy as jnp
from jax.experimental import pallas as pl
from jax.experimental.pallas import tpu as pltpu

NUM_CLS = 720
NUM_REG = 36
NUM_OUT = NUM_CLS + NUM_REG
FEAT_CH = 256


def _body(x_ref, w_ref, b_ref, cls_ref, reg_ref):
    x = x_ref[0]
    y = jax.lax.dot_general(
        w_ref[...], x,
        dimension_numbers=(((1,), (0,)), ((), ())),
        preferred_element_type=jnp.float32,
    ) + b_ref[...]
    cls_ref[0] = y[:NUM_CLS]
    reg_ref[0] = y[NUM_CLS:]


def kernel(feats, W_cls, b_cls, W_reg, b_reg):
    n, c, h, w = feats.shape
    hw = h * w
    x = feats.reshape(n, c, hw)
    W = jnp.concatenate([W_cls, W_reg], axis=0)
    b = jnp.concatenate([b_cls, b_reg], axis=0).reshape(NUM_OUT, 1)

    blk = 512
    nt = hw // blk

    cls_out, reg_out = pl.pallas_call(
        _body,
        grid=(n, nt),
        in_specs=[
            pl.BlockSpec((1, c, blk), lambda i, j: (i, 0, j)),
            pl.BlockSpec((NUM_OUT, c), lambda i, j: (0, 0)),
            pl.BlockSpec((NUM_OUT, 1), lambda i, j: (0, 0)),
        ],
        out_specs=[
            pl.BlockSpec((1, NUM_CLS, blk), lambda i, j: (i, 0, j)),
            pl.BlockSpec((1, NUM_REG, blk), lambda i, j: (i, 0, j)),
        ],
        out_shape=[
            jax.ShapeDtypeStruct((n, NUM_CLS, hw), jnp.float32),
            jax.ShapeDtypeStruct((n, NUM_REG, hw), jnp.float32),
        ],
        compiler_params=pltpu.CompilerParams(
            dimension_semantics=("parallel", "parallel"),
        ),
    )(x, W, b)

    return (cls_out.reshape(n, NUM_CLS, h, w), reg_out.reshape(n, NUM_REG, h, w))


# speedup vs baseline: 1.1694x; 1.1694x over previous
import jax
import jax.numpy as jnp
from jax.experimental import pallas as pl
from jax.experimental.pallas import tpu as pltpu

NUM_CLS = 720
NUM_REG = 36
NUM_OUT = NUM_CLS + NUM_REG
FEAT_CH = 256


def _body(x_ref, w_ref, b_ref, cls_ref, reg_ref):
    x = x_ref[0]
    y = jax.lax.dot_general(
        w_ref[...], x,
        dimension_numbers=(((1,), (0,)), ((), ())),
        preferred_element_type=jnp.float32,
    ) + b_ref[...]
    cls_ref[0] = y[:NUM_CLS]
    reg_ref[0] = y[NUM_CLS:]


def kernel(feats, W_cls, b_cls, W_reg, b_reg):
    n, c, h, w = feats.shape
    hw = h * w
    x = feats.reshape(n, c, hw)
    W = jnp.concatenate([W_cls, W_reg], axis=0)
    b = jnp.concatenate([b_cls, b_reg], axis=0).reshape(NUM_OUT, 1)

    blk = 4096
    nt = hw // blk

    cls_out, reg_out = pl.pallas_call(
        _body,
        grid=(n, nt),
        in_specs=[
            pl.BlockSpec((1, c, blk), lambda i, j: (i, 0, j)),
            pl.BlockSpec((NUM_OUT, c), lambda i, j: (0, 0)),
            pl.BlockSpec((NUM_OUT, 1), lambda i, j: (0, 0)),
        ],
        out_specs=[
            pl.BlockSpec((1, NUM_CLS, blk), lambda i, j: (i, 0, j)),
            pl.BlockSpec((1, NUM_REG, blk), lambda i, j: (i, 0, j)),
        ],
        out_shape=[
            jax.ShapeDtypeStruct((n, NUM_CLS, hw), jnp.float32),
            jax.ShapeDtypeStruct((n, NUM_REG, hw), jnp.float32),
        ],
        compiler_params=pltpu.CompilerParams(
            dimension_semantics=("parallel", "parallel"),
        ),
    )(x, W, b)

    return (cls_out.reshape(n, NUM_CLS, h, w), reg_out.reshape(n, NUM_REG, h, w))


# speedup vs baseline: 1.1741x; 1.0040x over previous
import jax
import jax.numpy as jnp
from jax.experimental import pallas as pl
from jax.experimental.pallas import tpu as pltpu

NUM_CLS = 720
NUM_REG = 36
NUM_OUT = NUM_CLS + NUM_REG
FEAT_CH = 256


def _body(x_ref, w_ref, b_ref, cls_ref, reg_ref):
    x = x_ref[0].astype(jnp.bfloat16)
    y = jax.lax.dot_general(
        w_ref[...].astype(jnp.bfloat16), x,
        dimension_numbers=(((1,), (0,)), ((), ())),
        preferred_element_type=jnp.float32,
    ) + b_ref[...]
    cls_ref[0] = y[:NUM_CLS]
    reg_ref[0] = y[NUM_CLS:]


def kernel(feats, W_cls, b_cls, W_reg, b_reg):
    n, c, h, w = feats.shape
    hw = h * w
    x = feats.reshape(n, c, hw)
    W = jnp.concatenate([W_cls, W_reg], axis=0)
    b = jnp.concatenate([b_cls, b_reg], axis=0).reshape(NUM_OUT, 1)

    blk = 4096
    nt = hw // blk

    cls_out, reg_out = pl.pallas_call(
        _body,
        grid=(n, nt),
        in_specs=[
            pl.BlockSpec((1, c, blk), lambda i, j: (i, 0, j)),
            pl.BlockSpec((NUM_OUT, c), lambda i, j: (0, 0)),
            pl.BlockSpec((NUM_OUT, 1), lambda i, j: (0, 0)),
        ],
        out_specs=[
            pl.BlockSpec((1, NUM_CLS, blk), lambda i, j: (i, 0, j)),
            pl.BlockSpec((1, NUM_REG, blk), lambda i, j: (i, 0, j)),
        ],
        out_shape=[
            jax.ShapeDtypeStruct((n, NUM_CLS, hw), jnp.float32),
            jax.ShapeDtypeStruct((n, NUM_REG, hw), jnp.float32),
        ],
        compiler_params=pltpu.CompilerParams(
            dimension_semantics=("parallel", "parallel"),
        ),
    )(x, W, b)

    return (cls_out.reshape(n, NUM_CLS, h, w), reg_out.reshape(n, NUM_REG, h, w))
